# baseline (speedup 1.0000x reference)
import functools
import jax
import jax.numpy as jnp
from jax import lax
from jax.experimental import pallas as pl
from jax.experimental.pallas import tpu as pltpu
from jax.experimental.pallas import tpu_sc as plsc

NU = 50000
NBD = 20000
NIT = 40000
NU_P = 51200
NBD_P = 20480
NIT_P = 40960
NDEG = NU_P + NIT_P
DH = 32
BATCH = 4096
L = 16
NT = 16
G = 128
NI = 8
EB = G * NI
FR = 64
CH = 360
FB = 16
DUMP = NU_P
UB_E = 1000000
UI_E = 1500000
BI_E = 500000


def _span(E):
    nblk = -(-E // EB)
    out = -(-nblk // NT)
    return out, out * NT * EB


UB_OUT, UB_SPAN = _span(UB_E)
UI_OUT, UI_SPAN = _span(UI_E)
BI_OUT, BI_SPAN = _span(BI_E)
UB_PAD = UB_E + UB_SPAN
UI_PAD = UI_E + UI_SPAN
BI_PAD = BI_SPAN


def _newton_rsqrt(d):
    bits = lax.bitcast_convert_type(d, jnp.int32)
    y = lax.bitcast_convert_type(
        jnp.int32(0x5F3759DF) - lax.shift_right_logical(bits, 1), jnp.float32)
    for _ in range(3):
        y = y * (jnp.float32(1.5) - jnp.float32(0.5) * d * y * y)
    return y


def _newton_recip(x):
    bits = lax.bitcast_convert_type(x, jnp.int32)
    y = lax.bitcast_convert_type(jnp.int32(0x7EF311C3) - bits, jnp.float32)
    for _ in range(4):
        y = y * (jnp.float32(2.0) - x * y)
    return y


def _body(ub_r, ub_c, ui_r, ui_c, bi_r, bi_c,
          ub_x0, ui_x0, users, bundles_flat, zrows, zflat,
          pred,
          ub_x1, ui_x1, ub_m, ui_m, bi_o, ub_x0p, ui_x0p, ub_x1p, ui_x1p,
          s_tab,
          acc, deg, ebr, ebc, gidx, sidx, gbuf, gbuf2, obuf, dbuf,
          f0, f1, f2, sbuf,
          uidx, bidx, ug_a, ug_b, bg_a, bg_b, pbuf, sem, sem2):
    h = lax.axis_index("c")
    t = lax.axis_index("s")
    third = jnp.float32(1.0 / 3.0)
    eps = jnp.float32(1e-8)
    zero16 = jnp.zeros((L,), jnp.float32)
    one16 = jnp.full((L,), 1.0, jnp.float32)
    iota = lax.iota(jnp.int32, L)

    def deg_half(er, e0, E, OUT, row_shift):
        @pl.loop(0, OUT)
        def _(o):
            b = o * NT + t
            eoff = e0 + b * EB
            pltpu.sync_copy(er.at[pl.ds(eoff, EB)], ebr)

            @pl.loop(0, NI)
            def _(j):
                @pl.loop(0, G // L)
                def _(k):
                    off = j * G + k * L
                    lim = E - (b * EB + off)
                    m = iota < lim
                    r = ebr[pl.ds(off, L)]
                    sidx[j, pl.ds(k * L, L)] = jnp.where(m, r + row_shift, 0)
                    obuf[pl.ds(k * L, L)] = jnp.where(m, one16, zero16)

                pltpu.sync_copy(obuf, deg.at[sidx.at[j]], add=True)

    def make_s(halves, n_deg, kind):
        pltpu.sync_copy(zflat, deg.at[pl.ds(t * (NDEG // NT), NDEG // NT)])
        plsc.subcore_barrier()
        for (er, e0, E, OUT, row_shift) in halves:
            deg_half(er, e0, E, OUT, row_shift)
        plsc.subcore_barrier()

        @pl.loop(0, n_deg // NT // CH)
        def _(i):
            start = t * (n_deg // NT) + i * CH
            pltpu.sync_copy(deg.at[pl.ds(start, CH)], dbuf)

            @pl.loop(0, CH // L)
            def _(v):
                d = dbuf[pl.ds(v * L, L)]
                if kind == "sym":
                    sq = d * _newton_rsqrt(d)
                    s = _newton_recip(sq + eps)
                else:
                    s = _newton_recip(d + eps)
                dbuf[pl.ds(v * L, L)] = s

            pltpu.sync_copy(dbuf, s_tab.at[h].at[pl.ds(start, CH)])
        plsc.subcore_barrier()

    def scale_rows(src, dst, n_rows):
        rpt = n_rows // NT

        @pl.loop(0, rpt // FR)
        def _(i):
            r = t * rpt + i * FR
            pltpu.sync_copy(src.at[h].at[pl.ds(r, FR)], f0)
            pltpu.sync_copy(s_tab.at[h].at[pl.ds(r, FR)], sbuf)

            @pl.loop(0, FR // L)
            def _(k):
                sv = sbuf[pl.ds(k * L, L)]
                for e2 in range(L):
                    rr = k * L + e2
                    s = sv[e2]
                    f0[rr, pl.ds(0, L)] = f0[rr, pl.ds(0, L)] * s
                    f0[rr, pl.ds(L, L)] = f0[rr, pl.ds(L, L)] * s

            pltpu.sync_copy(f0, dst.at[h].at[pl.ds(r, FR)])

    def spmm_pass(er, ec, e0, E, OUT, src, row_base, col_off, slab,
                  flush):
        slab_pt = slab // NT
        r0 = t * slab_pt

        pltpu.sync_copy(zrows.at[pl.ds(0, slab_pt)],
                        acc.at[pl.ds(r0, slab_pt)])
        plsc.subcore_barrier()

        def prep(b, j):
            @pl.loop(0, G // L)
            def _(k):
                off = j * G + k * L
                lim = E - (b * EB + off)
                m = iota < lim
                c = ebc[pl.ds(off, L)]
                r = ebr[pl.ds(off, L)]
                gidx[j, pl.ds(k * L, L)] = jnp.where(m, c + col_off, 0)
                sidx[j, pl.ds(k * L, L)] = jnp.where(m, r - row_base, DUMP)

        def fire(j, buf, sm):
            pltpu.async_copy(src.at[h].at[gidx.at[j]], buf, sm)

        def drain(j, buf, sm):
            pltpu.make_async_copy(src.at[h].at[gidx.at[j]], buf, sm).wait()

        @pl.loop(0, OUT)
        def _(o):
            b = o * NT + t
            eoff = e0 + b * EB
            pltpu.sync_copy(er.at[pl.ds(eoff, EB)], ebr)
            pltpu.sync_copy(ec.at[pl.ds(eoff, EB)], ebc)

            prep(b, 0)
            fire(0, gbuf, sem)

            @pl.loop(0, NI // 2)
            def _(j2):
                ja = 2 * j2
                jb = ja + 1
                prep(b, jb)
                fire(jb, gbuf2, sem2)
                drain(ja, gbuf, sem)
                pltpu.sync_copy(gbuf, acc.at[sidx.at[ja]], add=True)

                @pl.when(j2 < NI // 2 - 1)
                def _():
                    prep(b, ja + 2)
                    fire(ja + 2, gbuf, sem)

                drain(jb, gbuf2, sem2)
                pltpu.sync_copy(gbuf2, acc.at[sidx.at[jb]], add=True)

        plsc.subcore_barrier()
        flush(slab_pt, r0)
        plsc.subcore_barrier()

    def flush_l1(dst_raw, dst_scaled, row_off):
        def go(slab_pt, r0):
            @pl.loop(0, slab_pt // FR)
            def _(i):
                r = r0 + i * FR
                pltpu.sync_copy(acc.at[pl.ds(r, FR)], f0)
                pltpu.sync_copy(s_tab.at[h].at[pl.ds(row_off + r, FR)], sbuf)

                @pl.loop(0, FR // L)
                def _(k):
                    sv = sbuf[pl.ds(k * L, L)]
                    for e2 in range(L):
                        rr = k * L + e2
                        s = sv[e2]
                        a = f0[rr, pl.ds(0, L)] * s
                        bqq = f0[rr, pl.ds(L, L)] * s
                        f0[rr, pl.ds(0, L)] = a
                        f0[rr, pl.ds(L, L)] = bqq
                        f1[rr, pl.ds(0, L)] = a * s
                        f1[rr, pl.ds(L, L)] = bqq * s

                pltpu.sync_copy(f0, dst_raw.at[h].at[pl.ds(row_off + r, FR)])
                pltpu.sync_copy(f1, dst_scaled.at[h].at[pl.ds(row_off + r, FR)])
        return go

    def flush_mean(x0, x1, dst, row_off):
        def go(slab_pt, r0):
            @pl.loop(0, slab_pt // FR)
            def _(i):
                r = r0 + i * FR
                pltpu.sync_copy(acc.at[pl.ds(r, FR)], f0)
                pltpu.sync_copy(x0.at[h].at[pl.ds(row_off + r, FR)], f1)
                pltpu.sync_copy(x1.at[h].at[pl.ds(row_off + r, FR)], f2)
                pltpu.sync_copy(s_tab.at[h].at[pl.ds(row_off + r, FR)], sbuf)

                @pl.loop(0, FR // L)
                def _(k):
                    sv = sbuf[pl.ds(k * L, L)]
                    for e2 in range(L):
                        rr = k * L + e2
                        s = sv[e2]
                        a = (f0[rr, pl.ds(0, L)] * s + f1[rr, pl.ds(0, L)]
                             + f2[rr, pl.ds(0, L)]) * third
                        bqq = (f0[rr, pl.ds(L, L)] * s + f1[rr, pl.ds(L, L)]
                               + f2[rr, pl.ds(L, L)]) * third
                        f0[rr, pl.ds(0, L)] = a
                        f0[rr, pl.ds(L, L)] = bqq

                pltpu.sync_copy(f0, dst.at[h].at[pl.ds(row_off + r, FR)])
        return go

    def flush_bi(dst):
        def go(slab_pt, r0):
            @pl.loop(0, slab_pt // FR)
            def _(i):
                r = r0 + i * FR
                pltpu.sync_copy(acc.at[pl.ds(r, FR)], f0)
                pltpu.sync_copy(s_tab.at[h].at[pl.ds(r, FR)], sbuf)

                @pl.loop(0, FR // L)
                def _(k):
                    sv = sbuf[pl.ds(k * L, L)]
                    for e2 in range(L):
                        rr = k * L + e2
                        s = sv[e2]
                        f0[rr, pl.ds(0, L)] = f0[rr, pl.ds(0, L)] * s
                        f0[rr, pl.ds(L, L)] = f0[rr, pl.ds(L, L)] * s

                pltpu.sync_copy(f0, dst.at[h].at[pl.ds(r, FR)])
        return go

    SH = NU_P - NU

    make_s([(ui_r, 0, UI_E, UI_OUT, 0), (ui_r, UI_E, UI_E, UI_OUT, SH)],
           NDEG, "sym")
    scale_rows(ui_x0, ui_x0p, NU_P + NIT_P)
    plsc.subcore_barrier()
    spmm_pass(ui_r, ui_c, 0, UI_E, UI_OUT, ui_x0p, 0, SH, NU_P,
              flush_l1(ui_x1, ui_x1p, 0))
    spmm_pass(ui_r, ui_c, UI_E, UI_E, UI_OUT, ui_x0p, NU, 0, NIT_P,
              flush_l1(ui_x1, ui_x1p, NU_P))
    spmm_pass(ui_r, ui_c, 0, UI_E, UI_OUT, ui_x1p, 0, SH, NU_P,
              flush_mean(ui_x0, ui_x1, ui_m, 0))
    spmm_pass(ui_r, ui_c, UI_E, UI_E, UI_OUT, ui_x1p, NU, 0, NIT_P,
              flush_mean(ui_x0, ui_x1, ui_m, NU_P))
    make_s([(bi_r, 0, BI_E, BI_OUT, 0)], NDEG, "recip")
    spmm_pass(bi_r, bi_c, 0, BI_E, BI_OUT, ui_m, 0, NU_P, NBD_P,
              flush_bi(bi_o))
    make_s([(ub_r, 0, UB_E, UB_OUT, 0), (ub_r, UB_E, UB_E, UB_OUT, SH)],
           NDEG, "sym")
    scale_rows(ub_x0, ub_x0p, NU_P + NBD_P)
    plsc.subcore_barrier()
    spmm_pass(ub_r, ub_c, 0, UB_E, UB_OUT, ub_x0p, 0, SH, NU_P,
              flush_l1(ub_x1, ub_x1p, 0))
    spmm_pass(ub_r, ub_c, UB_E, UB_E, UB_OUT, ub_x0p, NU, 0, NBD_P,
              flush_l1(ub_x1, ub_x1p, NU_P))
    spmm_pass(ub_r, ub_c, 0, UB_E, UB_OUT, ub_x1p, 0, SH, NU_P,
              flush_mean(ub_x0, ub_x1, ub_m, 0))
    spmm_pass(ub_r, ub_c, UB_E, UB_E, UB_OUT, ub_x1p, NU, 0, NBD_P,
              flush_mean(ub_x0, ub_x1, ub_m, NU_P))

    nu_pt = BATCH // NT
    perms = [iota ^ (1 << p) for p in range(4)]

    def lane_sum(s):
        for pm in perms:
            s = s + s.at[pm].get(mode="promise_in_bounds")
        return s

    @pl.loop(0, nu_pt // FB)
    def _(i):
        u_off = t * nu_pt + i * FB
        pltpu.sync_copy(users.at[pl.ds(u_off, FB)], uidx)
        pltpu.sync_copy(bundles_flat.at[pl.ds(2 * u_off, 2 * FB)],
                        bidx.at[0])

        pltpu.sync_copy(ub_m.at[h].at[uidx], ug_a)
        pltpu.sync_copy(ui_m.at[h].at[uidx], ug_b)

        @pl.loop(0, 2 * FB // L)
        def _(k):
            bd = bidx[0, pl.ds(k * L, L)]
            bidx[1, pl.ds(k * L, L)] = bd + NU_P

        pltpu.sync_copy(ub_m.at[h].at[bidx.at[1]], bg_a)
        pltpu.sync_copy(bi_o.at[h].at[bidx.at[0]], bg_b)

        @pl.loop(0, 2 * FB // L)
        def _(g):
            predv = zero16
            for e2 in range(L):
                bq = g * (L // 2) + e2 // 2
                jj = e2 % 2
                u0 = ug_a[bq, pl.ds(0, L)]
                u1 = ug_a[bq, pl.ds(L, L)]
                u2 = ug_b[bq, pl.ds(0, L)]
                u3 = ug_b[bq, pl.ds(L, L)]
                s = (u0 * bg_a[2 * bq + jj, pl.ds(0, L)]
                     + u1 * bg_a[2 * bq + jj, pl.ds(L, L)]
                     + u2 * bg_b[2 * bq + jj, pl.ds(0, L)]
                     + u3 * bg_b[2 * bq + jj, pl.ds(L, L)])
                predv = jnp.where(iota == e2, lane_sum(s), predv)
            pbuf[pl.ds(g * L, L)] = predv

        pltpu.sync_copy(pbuf, pred.at[h].at[pl.ds(2 * u_off, 2 * FB)])


@jax.jit
def _run(ub_r, ub_c, ui_r, ui_c, bi_r, bi_c,
         ub_x0, ui_x0, users, bundles_flat):
    mesh = plsc.VectorSubcoreMesh(core_axis_name="c", subcore_axis_name="s")
    f32 = jnp.float32
    kfn = pl.kernel(
        _body,
        out_type=jax.ShapeDtypeStruct((2, BATCH * 2), f32),
        mesh=mesh,
        compiler_params=pltpu.CompilerParams(use_tc_tiling_on_sc=False),
        scratch_types=[
            pltpu.HBM((2, NU_P + NBD_P, DH), f32),
            pltpu.HBM((2, NU_P + NIT_P, DH), f32),
            pltpu.HBM((2, NU_P + NBD_P, DH), f32),
            pltpu.HBM((2, NU_P + NIT_P, DH), f32),
            pltpu.HBM((2, NBD_P, DH), f32),
            pltpu.HBM((2, NU_P + NBD_P, DH), f32),
            pltpu.HBM((2, NU_P + NIT_P, DH), f32),
            pltpu.HBM((2, NU_P + NBD_P, DH), f32),
            pltpu.HBM((2, NU_P + NIT_P, DH), f32),
            pltpu.HBM((2, NDEG), f32),
            pltpu.VMEM_SHARED((NU_P + 8, DH), f32),
            pltpu.VMEM_SHARED((NDEG,), f32),
            pltpu.VMEM((EB,), jnp.int32),
            pltpu.VMEM((EB,), jnp.int32),
            pltpu.VMEM((NI, G), jnp.int32),
            pltpu.VMEM((NI, G), jnp.int32),
            pltpu.VMEM((G, DH), f32),
            pltpu.VMEM((G, DH), f32),
            pltpu.VMEM((G,), f32),
            pltpu.VMEM((CH,), f32),
            pltpu.VMEM((FR, DH), f32),
            pltpu.VMEM((FR, DH), f32),
            pltpu.VMEM((FR, DH), f32),
            pltpu.VMEM((FR,), f32),
            pltpu.VMEM((FB,), jnp.int32),
            pltpu.VMEM((2, 2 * FB), jnp.int32),
            pltpu.VMEM((FB, DH), f32),
            pltpu.VMEM((FB, DH), f32),
            pltpu.VMEM((2 * FB, DH), f32),
            pltpu.VMEM((2 * FB, DH), f32),
            pltpu.VMEM((2 * FB,), f32),
            pltpu.SemaphoreType.DMA,
            pltpu.SemaphoreType.DMA,
        ],
    )
    zrows = jnp.zeros((NU_P // NT, DH), jnp.float32)
    zflat = jnp.zeros((NDEG // NT,), jnp.float32)
    return kfn(ub_r, ub_c, ui_r, ui_c, bi_r, bi_c,
               ub_x0, ui_x0, users, bundles_flat, zrows, zflat)


def _pad(x, n):
    return jnp.concatenate([x, jnp.zeros((n - x.shape[0],), x.dtype)])


def kernel(users_feature, bundles_feature, items_feature, ub_vals, ui_vals,
           bi_vals, ub_rows, ub_cols, ui_rows, ui_cols, bi_rows, bi_cols,
           users, bundles):
    def _halves(a):
        return jnp.stack([a[:, :DH], a[:, DH:]], axis=0)

    ub_x0 = jnp.zeros((2, NU_P + NBD_P, DH), jnp.float32)
    ub_x0 = ub_x0.at[:, :NU].set(_halves(users_feature))
    ub_x0 = ub_x0.at[:, NU_P:NU_P + NBD].set(_halves(bundles_feature))
    ui_x0 = jnp.zeros((2, NU_P + NIT_P, DH), jnp.float32)
    ui_x0 = ui_x0.at[:, :NU].set(_halves(users_feature))
    ui_x0 = ui_x0.at[:, NU_P:NU_P + NIT].set(_halves(items_feature))

    pred2 = _run(
        _pad(ub_rows, UB_PAD), _pad(ub_cols, UB_PAD),
        _pad(ui_rows, UI_PAD), _pad(ui_cols, UI_PAD),
        _pad(bi_rows, BI_PAD), _pad(bi_cols, BI_PAD),
        ub_x0, ui_x0, users, bundles.reshape(-1))

    p = (pred2[0] + pred2[1]).reshape(BATCH, 2)
    bpr = jnp.mean(jax.nn.softplus(p[:, 1] - p[:, 0]))
    return (bpr, jnp.zeros((1,), jnp.float32))
